# baseline (speedup 1.0000x reference)
import dataclasses
import functools

import jax
import jax.numpy as jnp
from jax import lax
from jax.experimental import pallas as pl
from jax.experimental.pallas import tpu as pltpu
from jax.experimental.pallas import tpu_sc as plsc

PAD = 0
B = 1024
L = 200
LANES = 16
LP = 256
V = 30522
VMAIN = 30464
W = 15232
TAILW = 128
NC = 2
NS = 16
NW = NC * NS
NSPLIT = 4
BC = B // NSPLIT
RPW = BC // NW
GR = 8
GROUPS = RPW // GR
NCHUNK = LP // LANES

_mesh = plsc.VectorSubcoreMesh(core_axis_name="c", subcore_axis_name="s")

_cp = pltpu.CompilerParams()
if "needs_layout_passes" in pltpu.CompilerParams.__dataclass_fields__:
    _cp = dataclasses.replace(_cp, needs_layout_passes=False)
if "use_tc_tiling_on_sc" in pltpu.CompilerParams.__dataclass_fields__:
    _cp = dataclasses.replace(_cp, use_tc_tiling_on_sc=True)


@functools.partial(
    pl.kernel,
    out_type=(
        jax.ShapeDtypeStruct((BC, V), jnp.float32),
        jax.ShapeDtypeStruct((BC, TAILW), jnp.float32),
    ),
    mesh=_mesh,
    scratch_types=[
        pltpu.VMEM((GR, LP), jnp.int32),
        pltpu.VMEM((GR, W), jnp.float32),
        pltpu.VMEM((GR, TAILW), jnp.float32),
    ],
    compiler_params=_cp,
)
def _bow(ids_hbm, out_hbm, tail_hbm, ids_v, buf, tailbuf):
    wid = lax.axis_index("s") * NC + lax.axis_index("c")
    base = wid * RPW

    zeros_f = jnp.zeros((LANES,), jnp.float32)
    zeros_i = jnp.zeros((LANES,), jnp.int32)

    for r in range(GR):
        @pl.loop(0, W, step=LANES)
        def _(i, r=r):
            buf[r, pl.ds(i, LANES)] = zeros_f

        @pl.loop(0, TAILW, step=LANES)
        def _(i, r=r):
            tailbuf[r, pl.ds(i, LANES)] = zeros_f

    @pl.loop(0, GROUPS)
    def _(g):
        rg = base + g * GR
        pltpu.sync_copy(ids_hbm.at[pl.ds(rg, GR)], ids_v)

        for lo in (0, W):
            hi = lo + W

            @pl.loop(0, GR)
            def _(r8, lo=lo, hi=hi):
                r8v = jnp.full((LANES,), r8, jnp.int32)
                carry = zeros_i
                for c in range(NCHUNK):
                    ids16 = ids_v[r8, pl.ds(c * LANES, LANES)]
                    is_pad = ids16 == PAD
                    cum = plsc.cumsum(is_pad.astype(jnp.int32))
                    valid = (cum + carry) == 0
                    carry = carry + plsc.all_reduce_population_count(is_pad)
                    cnt, last = plsc.scan_count(ids16, mask=valid)
                    sel = last & valid
                    cntf = cnt.astype(jnp.float32)
                    m = sel & (ids16 >= lo) & (ids16 < hi)
                    rel = jnp.where(m, ids16 - lo, 0)
                    plsc.addupdate_scatter(buf, [r8v, rel], cntf, mask=m)
                    if hi == VMAIN:
                        mt = sel & (ids16 >= VMAIN)
                        relt = jnp.where(mt, ids16 - VMAIN, 0)
                        plsc.addupdate_scatter(
                            tailbuf, [r8v, relt], cntf, mask=mt
                        )

            pltpu.sync_copy(buf, out_hbm.at[pl.ds(rg, GR), pl.ds(lo, W)])

            @pl.loop(0, GR)
            def _(r8, lo=lo, hi=hi):
                r8v = jnp.full((LANES,), r8, jnp.int32)
                for c in range(NCHUNK):
                    ids16 = ids_v[r8, pl.ds(c * LANES, LANES)]
                    inh = (ids16 >= lo) & (ids16 < hi)
                    rel = jnp.where(inh, ids16 - lo, 0)
                    plsc.store_scatter(buf, [r8v, rel], zeros_f, mask=inh)

        pltpu.sync_copy(tailbuf, tail_hbm.at[pl.ds(rg, GR)])

        @pl.loop(0, GR)
        def _(r8):
            r8v = jnp.full((LANES,), r8, jnp.int32)
            for c in range(NCHUNK):
                ids16 = ids_v[r8, pl.ds(c * LANES, LANES)]
                mt = ids16 >= VMAIN
                relt = jnp.where(mt, ids16 - VMAIN, 0)
                plsc.store_scatter(tailbuf, [r8v, relt], zeros_f, mask=mt)


def _patch_tail(main, tail):
    def body(t_ref, m_ref, o_ref):
        o_ref[...] = t_ref[...]

    return pl.pallas_call(
        body,
        grid=(1,),
        in_specs=[
            pl.BlockSpec((BC, TAILW), lambda i: (0, 0)),
            pl.BlockSpec(memory_space=pl.ANY),
        ],
        out_specs=pl.BlockSpec((BC, TAILW), lambda i: (0, VMAIN // TAILW)),
        out_shape=jax.ShapeDtypeStruct((BC, V), jnp.float32),
        input_output_aliases={1: 0},
    )(tail, main)


def kernel(input_ids):
    ids = jnp.pad(input_ids, ((0, 0), (0, LP - L)))
    parts = []
    for k in range(NSPLIT):
        main, tail = _bow(lax.slice(ids, (k * BC, 0), ((k + 1) * BC, LP)))
        parts.append(_patch_tail(main, tail))
    return jnp.concatenate(parts, axis=0)


# speedup vs baseline: 1.8186x; 1.8186x over previous
import dataclasses
import functools

import jax
import jax.numpy as jnp
from jax import lax
from jax.experimental import pallas as pl
from jax.experimental.pallas import tpu as pltpu
from jax.experimental.pallas import tpu_sc as plsc

PAD = 0
B = 1024
L = 200
LANES = 16
LP = 256
V = 30522
VMAIN = 30464
NQ = 7
QW = 4352
TAILW = 128
NC = 2
NS = 16
NW = NC * NS
RPW = B // NW
GR = 8
GROUPS = RPW // GR
NCHUNK = LP // LANES

_mesh = plsc.VectorSubcoreMesh(core_axis_name="c", subcore_axis_name="s")

_cp = pltpu.CompilerParams()
if "needs_layout_passes" in pltpu.CompilerParams.__dataclass_fields__:
    _cp = dataclasses.replace(_cp, needs_layout_passes=False)
if "use_tc_tiling_on_sc" in pltpu.CompilerParams.__dataclass_fields__:
    _cp = dataclasses.replace(_cp, use_tc_tiling_on_sc=True)


@functools.partial(
    pl.kernel,
    out_type=(
        jax.ShapeDtypeStruct((B, V), jnp.float32),
        jax.ShapeDtypeStruct((B, TAILW), jnp.float32),
    ),
    mesh=_mesh,
    scratch_types=[
        pltpu.VMEM((GR, LP), jnp.int32),
        pltpu.VMEM((GR, QW), jnp.float32),
        pltpu.VMEM((GR, QW), jnp.float32),
        pltpu.VMEM((GR, LP), jnp.float32),
        pltpu.VMEM((GR, LP), jnp.int32),
        pltpu.VMEM((GR, TAILW), jnp.float32),
        pltpu.SemaphoreType.DMA,
        pltpu.SemaphoreType.DMA,
    ],
    compiler_params=_cp,
)
def _bow(ids_hbm, out_hbm, tail_hbm, ids_v, buf0, buf1, cnt_v, sel_v,
         tailbuf, sem0, sem1):
    wid = lax.axis_index("s") * NC + lax.axis_index("c")
    base = wid * RPW

    zeros_f = jnp.zeros((LANES,), jnp.float32)
    zeros_i = jnp.zeros((LANES,), jnp.int32)
    bufs = (buf0, buf1)
    sems = (sem0, sem1)

    for bq in bufs:
        for r in range(GR):
            @pl.loop(0, QW, step=LANES)
            def _(i, r=r, bq=bq):
                bq[r, pl.ds(i, LANES)] = zeros_f

    for r in range(GR):
        @pl.loop(0, TAILW, step=LANES)
        def _(i, r=r):
            tailbuf[r, pl.ds(i, LANES)] = zeros_f

    @pl.loop(0, GROUPS)
    def _(g):
        rg = base + g * GR
        pltpu.sync_copy(ids_hbm.at[pl.ds(rg, GR)], ids_v)

        for q in range(NQ):
            lo = q * QW
            hi = lo + QW
            bq, sem = bufs[q % 2], sems[q % 2]

            if q >= 2:
                plo = (q - 2) * QW
                pltpu.make_async_copy(
                    bq, out_hbm.at[pl.ds(rg, GR), pl.ds(plo, QW)], sem
                ).wait()

                @pl.loop(0, GR)
                def _(r8, plo=plo, bq=bq):
                    r8v = jnp.full((LANES,), r8, jnp.int32)
                    for c in range(NCHUNK):
                        ids16 = ids_v[r8, pl.ds(c * LANES, LANES)]
                        inq = (ids16 >= plo) & (ids16 < plo + QW)
                        rel = jnp.where(inq, ids16 - plo, 0)
                        plsc.store_scatter(bq, [r8v, rel], zeros_f, mask=inq)

            @pl.loop(0, GR)
            def _(r8, q=q, lo=lo, hi=hi, bq=bq):
                r8v = jnp.full((LANES,), r8, jnp.int32)
                carry = zeros_i
                for c in range(NCHUNK):
                    sl = pl.ds(c * LANES, LANES)
                    ids16 = ids_v[r8, sl]
                    if q == 0:
                        is_pad = ids16 == PAD
                        cum = plsc.cumsum(is_pad.astype(jnp.int32))
                        valid = (cum + carry) == 0
                        carry = carry + plsc.all_reduce_population_count(
                            is_pad
                        )
                        cnt, last = plsc.scan_count(ids16, mask=valid)
                        sel = last & valid
                        cntf = cnt.astype(jnp.float32)
                        cnt_v[r8, sl] = cntf
                        sel_v[r8, sl] = sel.astype(jnp.int32)
                        mt = sel & (ids16 >= VMAIN)
                        relt = jnp.where(mt, ids16 - VMAIN, 0)
                        plsc.addupdate_scatter(
                            tailbuf, [r8v, relt], cntf, mask=mt
                        )
                    else:
                        cntf = cnt_v[r8, sl]
                        sel = sel_v[r8, sl] != 0
                    m = sel & (ids16 >= lo) & (ids16 < hi)
                    rel = jnp.where(m, ids16 - lo, 0)
                    plsc.addupdate_scatter(bq, [r8v, rel], cntf, mask=m)

            pltpu.async_copy(
                bq, out_hbm.at[pl.ds(rg, GR), pl.ds(lo, QW)], sem
            )

        for q in (NQ - 2, NQ - 1):
            lo = q * QW
            bq, sem = bufs[q % 2], sems[q % 2]
            pltpu.make_async_copy(
                bq, out_hbm.at[pl.ds(rg, GR), pl.ds(lo, QW)], sem
            ).wait()

            @pl.loop(0, GR)
            def _(r8, lo=lo, bq=bq):
                r8v = jnp.full((LANES,), r8, jnp.int32)
                for c in range(NCHUNK):
                    ids16 = ids_v[r8, pl.ds(c * LANES, LANES)]
                    inq = (ids16 >= lo) & (ids16 < lo + QW)
                    rel = jnp.where(inq, ids16 - lo, 0)
                    plsc.store_scatter(bq, [r8v, rel], zeros_f, mask=inq)

        pltpu.sync_copy(tailbuf, tail_hbm.at[pl.ds(rg, GR)])

        @pl.loop(0, GR)
        def _(r8):
            r8v = jnp.full((LANES,), r8, jnp.int32)
            for c in range(NCHUNK):
                ids16 = ids_v[r8, pl.ds(c * LANES, LANES)]
                mt = ids16 >= VMAIN
                relt = jnp.where(mt, ids16 - VMAIN, 0)
                plsc.store_scatter(tailbuf, [r8v, relt], zeros_f, mask=mt)


def kernel(input_ids):
    ids = jnp.pad(input_ids, ((0, 0), (0, LP - L)))
    main, tail = _bow(ids)
    tail58 = lax.slice(tail, (0, 0), (B, V - VMAIN))
    return lax.dynamic_update_slice(main, tail58, (0, VMAIN))


# speedup vs baseline: 1.8376x; 1.0104x over previous
import dataclasses
import functools

import jax
import jax.numpy as jnp
from jax import lax
from jax.experimental import pallas as pl
from jax.experimental.pallas import tpu as pltpu
from jax.experimental.pallas import tpu_sc as plsc

PAD = 0
B = 1024
L = 200
LANES = 16
LP = 256
V = 30522
VMAIN = 30464
W = 15232
TAILW = 128
NC = 2
NS = 16
NW = NC * NS
RPW = B // NW
GR = 8
GROUPS = RPW // GR
NCHUNK = LP // LANES

_mesh = plsc.VectorSubcoreMesh(core_axis_name="c", subcore_axis_name="s")

_cp = pltpu.CompilerParams()
if "needs_layout_passes" in pltpu.CompilerParams.__dataclass_fields__:
    _cp = dataclasses.replace(_cp, needs_layout_passes=False)
if "use_tc_tiling_on_sc" in pltpu.CompilerParams.__dataclass_fields__:
    _cp = dataclasses.replace(_cp, use_tc_tiling_on_sc=True)


@functools.partial(
    pl.kernel,
    out_type=(
        jax.ShapeDtypeStruct((B, V), jnp.float32),
        jax.ShapeDtypeStruct((B, TAILW), jnp.float32),
    ),
    mesh=_mesh,
    scratch_types=[
        pltpu.VMEM((GR, LP), jnp.int32),
        pltpu.VMEM((GR, W), jnp.float32),
        pltpu.VMEM((GR, TAILW), jnp.float32),
    ],
    compiler_params=_cp,
)
def _bow(ids_hbm, out_hbm, tail_hbm, ids_v, buf, tailbuf):
    wid = lax.axis_index("s") * NC + lax.axis_index("c")
    base = wid * RPW

    zeros_f = jnp.zeros((LANES,), jnp.float32)
    zeros_i = jnp.zeros((LANES,), jnp.int32)

    for r in range(GR):
        @pl.loop(0, W, step=LANES)
        def _(i, r=r):
            buf[r, pl.ds(i, LANES)] = zeros_f

        @pl.loop(0, TAILW, step=LANES)
        def _(i, r=r):
            tailbuf[r, pl.ds(i, LANES)] = zeros_f

    @pl.loop(0, GROUPS)
    def _(g):
        rg = base + g * GR
        pltpu.sync_copy(ids_hbm.at[pl.ds(rg, GR)], ids_v)

        for lo in (0, W):
            hi = lo + W

            @pl.loop(0, GR)
            def _(r8, lo=lo, hi=hi):
                r8v = jnp.full((LANES,), r8, jnp.int32)
                carry = zeros_i
                for c in range(NCHUNK):
                    ids16 = ids_v[r8, pl.ds(c * LANES, LANES)]
                    is_pad = ids16 == PAD
                    cum = plsc.cumsum(is_pad.astype(jnp.int32))
                    valid = (cum + carry) == 0
                    carry = carry + plsc.all_reduce_population_count(is_pad)
                    cnt, last = plsc.scan_count(ids16, mask=valid)
                    sel = last & valid
                    cntf = cnt.astype(jnp.float32)
                    m = sel & (ids16 >= lo) & (ids16 < hi)
                    rel = jnp.where(m, ids16 - lo, 0)
                    plsc.addupdate_scatter(buf, [r8v, rel], cntf, mask=m)
                    if hi == VMAIN:
                        mt = sel & (ids16 >= VMAIN)
                        relt = jnp.where(mt, ids16 - VMAIN, 0)
                        plsc.addupdate_scatter(
                            tailbuf, [r8v, relt], cntf, mask=mt
                        )

            pltpu.sync_copy(buf, out_hbm.at[pl.ds(rg, GR), pl.ds(lo, W)])

            @pl.loop(0, GR)
            def _(r8, lo=lo, hi=hi):
                r8v = jnp.full((LANES,), r8, jnp.int32)
                for c in range(NCHUNK):
                    ids16 = ids_v[r8, pl.ds(c * LANES, LANES)]
                    inh = (ids16 >= lo) & (ids16 < hi)
                    rel = jnp.where(inh, ids16 - lo, 0)
                    plsc.store_scatter(buf, [r8v, rel], zeros_f, mask=inh)

        pltpu.sync_copy(tailbuf, tail_hbm.at[pl.ds(rg, GR)])

        @pl.loop(0, GR)
        def _(r8):
            r8v = jnp.full((LANES,), r8, jnp.int32)
            for c in range(NCHUNK):
                ids16 = ids_v[r8, pl.ds(c * LANES, LANES)]
                mt = ids16 >= VMAIN
                relt = jnp.where(mt, ids16 - VMAIN, 0)
                plsc.store_scatter(tailbuf, [r8v, relt], zeros_f, mask=mt)


def kernel(input_ids):
    ids = jnp.pad(input_ids, ((0, 0), (0, LP - L)))
    main, tail = _bow(ids)
    tail58 = lax.slice(tail, (0, 0), (B, V - VMAIN))
    return lax.dynamic_update_slice(main, tail58, (0, VMAIN))
